# baseline (speedup 1.0000x reference)
import functools

import jax
import jax.numpy as jnp
from jax import lax
from jax.experimental import pallas as pl
from jax.experimental.pallas import tpu as pltpu
from jax.experimental.pallas import tpu_sc as plsc

UP = 2
VOL = 8
BK = 16384

_NW = 32
_CM = 512
_N = 200000
_NFULL = _N // _CM
_REM = _N - _NFULL * _CM
_REMP = 384
_NSTEP = (_NFULL + 1 + _NW - 1) // _NW


def _idx_body(i_ref, o_ref):
    x = i_ref[...]
    nk = BK // 128

    r4 = lax.broadcasted_iota(jnp.int32, (4, BK), 0)
    xs = jnp.where(r4 == 0, x, x * UP).reshape(4 * nk, 128).astype(jnp.float32)

    s_i = lax.broadcasted_iota(jnp.int32, (128, 8 * 128), 0)
    j_i = lax.broadcasted_iota(jnp.int32, (128, 8 * 128), 1)
    rep = (s_i == j_i // 8).astype(jnp.float32)

    y = lax.dot_general(
        xs, rep, (((1,), (0,)), ((), ())), preferred_element_type=jnp.float32
    )

    c1 = lax.broadcasted_iota(jnp.int32, (4, 1, 8 * 128), 0)
    v1 = lax.broadcasted_iota(jnp.int32, (4, 1, 8 * 128), 2) & 7
    bias = jnp.where(
        c1 == 0,
        0,
        jnp.where(c1 == 1, v1 >> 2, jnp.where(c1 == 2, v1 & 1, (v1 >> 1) & 1)),
    )
    z = y.astype(jnp.int32).reshape(4, nk, 8 * 128) + bias
    o_ref[...] = z.reshape(4, 8 * BK)


def _sc_feats(ft_hbm, tail_hbm, fp_hbm, in_v, out_v, insem, outsem):
    wid = lax.axis_index("s") * 2 + lax.axis_index("c")
    lane = lax.iota(jnp.int32, 16)
    i1b = lane >> 3

    def in_full(s):
        ch = wid + _NW * s
        m0 = pl.multiple_of(ch * _CM, _CM)
        return pltpu.make_async_copy(
            ft_hbm.at[:, pl.ds(m0, _CM)], in_v.at[s % 2], insem.at[s % 2]
        )

    def in_tail(s):
        return pltpu.make_async_copy(
            tail_hbm, in_v.at[s % 2, :, pl.ds(0, _REMP)], insem.at[s % 2]
        )

    def out_full(s):
        ch = wid + _NW * s
        o0 = pl.multiple_of(ch * (VOL * _CM), VOL * _CM)
        return pltpu.make_async_copy(
            out_v.at[s % 2], fp_hbm.at[:, pl.ds(o0, VOL * _CM)], outsem.at[s % 2]
        )

    def out_tail(s):
        return pltpu.make_async_copy(
            out_v.at[s % 2, :, pl.ds(0, VOL * _REM)],
            fp_hbm.at[:, pl.ds(VOL * _NFULL * _CM, VOL * _REM)],
            outsem.at[s % 2],
        )

    def on_chunk(s, fn_full, fn_tail):
        ch = wid + _NW * s

        @pl.when(ch < _NFULL)
        def _():
            fn_full(s)

        @pl.when(ch == _NFULL)
        def _():
            fn_tail(s)

    def interleave(s, cols):
        for c in range(VOL):
            i0 = 8 * (lane & 7) + c

            @plsc.parallel_loop(0, VOL * cols // 16, unroll=8)
            def _(g):
                gat = plsc.load_gather(in_v.at[s % 2], [i0, i1b + 2 * g])
                out_v[s % 2, c, pl.ds(16 * g, 16)] = gat

    on_chunk(0, lambda s: in_full(s).start(), lambda s: in_tail(s).start())
    for s in range(_NSTEP):
        if s + 1 < _NSTEP:
            on_chunk(s + 1, lambda t: in_full(t).start(), lambda t: in_tail(t).start())
        on_chunk(s, lambda t: in_full(t).wait(), lambda t: in_tail(t).wait())
        if s >= 2:
            on_chunk(s - 2, lambda t: out_full(t).wait(), lambda t: out_tail(t).wait())
        on_chunk(s, lambda t: interleave(t, _CM), lambda t: interleave(t, _REM))
        on_chunk(s, lambda t: out_full(t).start(), lambda t: out_tail(t).start())
    for s in (_NSTEP - 2, _NSTEP - 1):
        on_chunk(s, lambda t: out_full(t).wait(), lambda t: out_tail(t).wait())


@jax.jit
def kernel(features, indices):
    n, nch = features.shape

    it = indices.T
    g = (n + BK - 1) // BK
    nip = pl.pallas_call(
        _idx_body,
        grid=(g,),
        in_specs=[pl.BlockSpec((4, BK), lambda i: (0, i))],
        out_specs=pl.BlockSpec((4, VOL * BK), lambda i: (0, i)),
        out_shape=jax.ShapeDtypeStruct((4, VOL * n), jnp.int32),
        compiler_params=pltpu.CompilerParams(
            dimension_semantics=("arbitrary",),
        ),
    )(it)

    ft = features.T
    tail = jnp.pad(features[_NFULL * _CM :, :].T, ((0, 0), (0, _REMP - _REM)))
    mesh = plsc.VectorSubcoreMesh(core_axis_name="c", subcore_axis_name="s")
    fpp = functools.partial(
        pl.kernel,
        mesh=mesh,
        out_type=jax.ShapeDtypeStruct((VOL, n * VOL), features.dtype),
        scratch_types=[
            pltpu.VMEM((2, nch, _CM), features.dtype),
            pltpu.VMEM((2, VOL, VOL * _CM), features.dtype),
            pltpu.SemaphoreType.DMA((2,)),
            pltpu.SemaphoreType.DMA((2,)),
        ],
        compiler_params=pltpu.CompilerParams(needs_layout_passes=False),
    )(_sc_feats)(ft, tail)

    return fpp.T, nip.T


# speedup vs baseline: 1.1081x; 1.1081x over previous
import functools

import jax
import jax.numpy as jnp
from jax import lax
from jax.experimental import pallas as pl
from jax.experimental.pallas import tpu as pltpu
from jax.experimental.pallas import tpu_sc as plsc

UP = 2
VOL = 8
BK = 16384

_NW = 32
_CM = 256
_NBUF = 3
_N = 200000
_NFULL = _N // _CM
_REM = _N - _NFULL * _CM
_REMP = 128
_NSTEP = (_NFULL + 1 + _NW - 1) // _NW


def _idx_body(i_ref, o_ref):
    x = i_ref[...]
    nk = BK // 128

    r4 = lax.broadcasted_iota(jnp.int32, (4, BK), 0)
    xs = jnp.where(r4 == 0, x, x * UP).reshape(4 * nk, 128).astype(jnp.float32)

    s_i = lax.broadcasted_iota(jnp.int32, (128, 8 * 128), 0)
    j_i = lax.broadcasted_iota(jnp.int32, (128, 8 * 128), 1)
    rep = (s_i == j_i // 8).astype(jnp.float32)

    y = lax.dot_general(
        xs, rep, (((1,), (0,)), ((), ())), preferred_element_type=jnp.float32
    )

    c1 = lax.broadcasted_iota(jnp.int32, (4, 1, 8 * 128), 0)
    v1 = lax.broadcasted_iota(jnp.int32, (4, 1, 8 * 128), 2) & 7
    bias = jnp.where(
        c1 == 0,
        0,
        jnp.where(c1 == 1, v1 >> 2, jnp.where(c1 == 2, v1 & 1, (v1 >> 1) & 1)),
    )
    z = y.astype(jnp.int32).reshape(4, nk, 8 * 128) + bias
    o_ref[...] = z.reshape(4, 8 * BK)


def _sc_feats(ft_hbm, tail_hbm, fp_hbm, in_v, out_v, insem, outsem):
    wid = lax.axis_index("s") * 2 + lax.axis_index("c")
    lane = lax.iota(jnp.int32, 16)
    i1b = lane >> 3

    def in_full(s):
        ch = wid + _NW * s
        m0 = pl.multiple_of(ch * _CM, _CM)
        return pltpu.make_async_copy(
            ft_hbm.at[:, pl.ds(m0, _CM)], in_v.at[s % _NBUF], insem.at[s % _NBUF]
        )

    def in_tail(s):
        return pltpu.make_async_copy(
            tail_hbm, in_v.at[s % _NBUF, :, pl.ds(0, _REMP)], insem.at[s % _NBUF]
        )

    def out_full(s):
        ch = wid + _NW * s
        o0 = pl.multiple_of(ch * (VOL * _CM), VOL * _CM)
        return pltpu.make_async_copy(
            out_v.at[s % _NBUF], fp_hbm.at[:, pl.ds(o0, VOL * _CM)], outsem.at[s % _NBUF]
        )

    def out_tail(s):
        return pltpu.make_async_copy(
            out_v.at[s % _NBUF, :, pl.ds(0, VOL * _REM)],
            fp_hbm.at[:, pl.ds(VOL * _NFULL * _CM, VOL * _REM)],
            outsem.at[s % _NBUF],
        )

    def on_chunk(s, fn_full, fn_tail):
        ch = wid + _NW * s

        @pl.when(ch < _NFULL)
        def _():
            fn_full(s)

        @pl.when(ch == _NFULL)
        def _():
            fn_tail(s)

    def interleave(s, cols):
        for c in range(VOL):
            i0 = 8 * (lane & 7) + c

            @plsc.parallel_loop(0, VOL * cols // 16, unroll=8)
            def _(g):
                gat = plsc.load_gather(in_v.at[s % _NBUF], [i0, i1b + 2 * g])
                out_v[s % _NBUF, c, pl.ds(16 * g, 16)] = gat

    for p in range(_NBUF - 1):
        on_chunk(p, lambda s: in_full(s).start(), lambda s: in_tail(s).start())

    def step(s, carry):
        @pl.when(s + _NBUF - 1 < _NSTEP)
        def _():
            on_chunk(
                s + _NBUF - 1,
                lambda t: in_full(t).start(),
                lambda t: in_tail(t).start(),
            )

        on_chunk(s, lambda t: in_full(t).wait(), lambda t: in_tail(t).wait())

        @pl.when(s >= _NBUF)
        def _():
            on_chunk(
                s - _NBUF, lambda t: out_full(t).wait(), lambda t: out_tail(t).wait()
            )

        on_chunk(s, lambda t: interleave(t, _CM), lambda t: interleave(t, _REM))
        on_chunk(s, lambda t: out_full(t).start(), lambda t: out_tail(t).start())
        return carry

    lax.fori_loop(0, _NSTEP, step, 0)
    for s in range(_NSTEP - _NBUF, _NSTEP):
        on_chunk(s, lambda t: out_full(t).wait(), lambda t: out_tail(t).wait())


@jax.jit
def kernel(features, indices):
    n, nch = features.shape

    it = indices.T
    g = (n + BK - 1) // BK
    nip = pl.pallas_call(
        _idx_body,
        grid=(g,),
        in_specs=[pl.BlockSpec((4, BK), lambda i: (0, i))],
        out_specs=pl.BlockSpec((4, VOL * BK), lambda i: (0, i)),
        out_shape=jax.ShapeDtypeStruct((4, VOL * n), jnp.int32),
        compiler_params=pltpu.CompilerParams(
            dimension_semantics=("arbitrary",),
        ),
    )(it)

    ft = features.T
    tail = jnp.pad(features[_NFULL * _CM :, :].T, ((0, 0), (0, _REMP - _REM)))
    mesh = plsc.VectorSubcoreMesh(core_axis_name="c", subcore_axis_name="s")
    fpp = functools.partial(
        pl.kernel,
        mesh=mesh,
        out_type=jax.ShapeDtypeStruct((VOL, n * VOL), features.dtype),
        scratch_types=[
            pltpu.VMEM((_NBUF, nch, _CM), features.dtype),
            pltpu.VMEM((_NBUF, VOL, VOL * _CM), features.dtype),
            pltpu.SemaphoreType.DMA((_NBUF,)),
            pltpu.SemaphoreType.DMA((_NBUF,)),
        ],
        compiler_params=pltpu.CompilerParams(needs_layout_passes=False),
    )(_sc_feats)(ft, tail)

    return fpp.T, nip.T
